# baseline (speedup 1.0000x reference)
import jax
import jax.numpy as jnp
from jax import lax
from jax.experimental import pallas as pl
from jax.experimental.pallas import tpu as pltpu
from jax.experimental.pallas import tpu_sc as plsc

_NUM_CLASSES = 1000000
_FEAT_DIM = 64
_BATCH = 16384
_LAMBDA_C = 1.0

_NC = 2
_NS = 16
_NW = _NC * _NS
_LANES = 16
_SB = 512
_TAIL = (_NUM_CLASSES // _SB) * _SB
_NSB = _TAIL // _SB
_SB_PER_W = _NSB // _NW
_MAXPAIR = (_SB_PER_W + 2) // 2
_SCAN_GROUPS = _BATCH // _LANES
_STAGE = 16


def _k1_body(lab_hbm, centT_hbm, e_hbm,
             lab_all, blk_a, blk_b, srt, estage,
             cnt_s, off_s, cur_s, misc_s,
             sem_a, sem_b, sem_e):
    wid = lax.axis_index("s") * _NC + lax.axis_index("c")
    nsb = _SB_PER_W + jnp.where(wid == _NW - 1, 1, 0)
    lo = wid * (_SB_PER_W * _SB)
    rng = nsb * _SB
    iota = lax.iota(jnp.int32, _LANES)

    pltpu.sync_copy(lab_hbm, lab_all)

    def init_cnt(b, c):
        cnt_s[b] = 0
        return c
    lax.fori_loop(0, _SB_PER_W + 2, init_cnt, 0)

    def scan1(g, c):
        vec = lab_all[pl.ds(g * _LANES, _LANES)]
        rel = vec - lo
        match = (rel >= 0) & (rel < rng)
        mi = jnp.where(match, 1, 0)
        npos = plsc.all_reduce_population_count(match)

        @pl.when(npos[0] == 1)
        def _():
            sb = jnp.sum(jnp.where(match, rel, 0)) >> 9
            cnt_s[sb] = cnt_s[sb] + 1

        @pl.when(npos[0] > 1)
        def _():
            for l in range(_LANES):
                @pl.when(mi[l] == 1)
                def _():
                    sb = rel[l] >> 9
                    cnt_s[sb] = cnt_s[sb] + 1
        return c
    lax.fori_loop(0, _SCAN_GROUPS, scan1, 0)

    misc_s[0] = 0

    def prefix(b, c):
        v = misc_s[0]
        off_s[b] = v
        cur_s[b] = v
        misc_s[0] = v + cnt_s[b]
        return c
    lax.fori_loop(0, _SB_PER_W + 2, prefix, 0)

    lane0 = iota == 0

    def place_one(rel_s, bidx_s):
        sb = rel_s >> 9
        p = cur_s[sb]
        cur_s[sb] = p + 1
        plsc.store_scatter(
            srt, [jnp.full((_LANES,), p, jnp.int32)],
            jnp.full((_LANES,), rel_s * _BATCH + bidx_s, jnp.int32),
            mask=lane0)

    def scan2(g, c):
        vec = lab_all[pl.ds(g * _LANES, _LANES)]
        rel = vec - lo
        match = (rel >= 0) & (rel < rng)
        mi = jnp.where(match, 1, 0)
        npos = plsc.all_reduce_population_count(match)

        @pl.when(npos[0] == 1)
        def _():
            rel_s = jnp.sum(jnp.where(match, rel, 0))
            bidx_s = g * _LANES + jnp.sum(jnp.where(match, iota, 0))
            place_one(rel_s, bidx_s)

        @pl.when(npos[0] > 1)
        def _():
            for l in range(_LANES):
                @pl.when(mi[l] == 1)
                def _():
                    place_one(rel[l], g * _LANES + l)
        return c
    lax.fori_loop(0, _SCAN_GROUPS, scan2, 0)

    misc_s[1] = 0

    def fire(sb_local, buf, sem):
        return pltpu.async_copy(
            centT_hbm.at[:, pl.ds(lo + sb_local * _SB, _SB)], buf, sem)

    @pl.when(0 < nsb)
    def _():
        fire(0, blk_a, sem_a)

    @pl.when(1 < nsb)
    def _():
        fire(1, blk_b, sem_b)

    def extract(sb, buf):
        m0 = off_s[sb]
        m1 = off_s[sb + 1]

        def per_match(m, c, buf=buf, sb=sb):
            v = plsc.load_gather(srt, [jnp.full((_LANES,), m, jnp.int32)])
            val = v[0]
            rel = val // _BATCH
            b = val - rel * _BATCH
            col = rel & (_SB - 1)
            colv = jnp.full((_LANES,), col, jnp.int32)
            rc = misc_s[1]
            slot = rc & (_STAGE - 1)

            @pl.when(rc >= _STAGE)
            def _():
                pltpu.make_async_copy(
                    e_hbm.at[pl.ds(0, 1), :],
                    estage.at[pl.ds(_STAGE, 1), :], sem_e).wait()

            for gg in range(_FEAT_DIM // _LANES):
                dvec = iota + gg * _LANES
                vals = plsc.load_gather(buf, [dvec, colv])
                estage[slot, pl.ds(gg * _LANES, _LANES)] = vals
            pltpu.async_copy(estage.at[pl.ds(slot, 1), :],
                             e_hbm.at[pl.ds(b, 1), :], sem_e)
            misc_s[1] = rc + 1
            return c

        lax.fori_loop(m0, m1, per_match, 0)

    def pair_step(tp, c):
        sb0 = tp * 2
        sb1 = sb0 + 1

        @pl.when(sb0 < nsb)
        def _():
            pltpu.make_async_copy(
                centT_hbm.at[:, pl.ds(0, _SB)], blk_a, sem_a).wait()
            extract(sb0, blk_a)

            @pl.when(sb0 + 2 < nsb)
            def _():
                fire(sb0 + 2, blk_a, sem_a)

        @pl.when(sb1 < nsb)
        def _():
            pltpu.make_async_copy(
                centT_hbm.at[:, pl.ds(0, _SB)], blk_b, sem_b).wait()
            extract(sb1, blk_b)

            @pl.when(sb1 + 2 < nsb)
            def _():
                fire(sb1 + 2, blk_b, sem_b)
        return c

    lax.fori_loop(0, _MAXPAIR, pair_step, 0)

    def drain(i, c):
        pltpu.make_async_copy(
            e_hbm.at[pl.ds(0, 1), :],
            estage.at[pl.ds(_STAGE, 1), :], sem_e).wait()
        return c
    lax.fori_loop(0, jnp.minimum(misc_s[1], _STAGE), drain, 0)


def _k1(labels, centersT):
    mesh = plsc.VectorSubcoreMesh(core_axis_name="c", subcore_axis_name="s")
    return pl.kernel(
        _k1_body,
        mesh=mesh,
        compiler_params=pltpu.CompilerParams(needs_layout_passes=False,
                                             use_tc_tiling_on_sc=True),
        out_type=jax.ShapeDtypeStruct((_BATCH, _FEAT_DIM), jnp.float32),
        scratch_types=[
            pltpu.VMEM((_BATCH,), jnp.int32),
            pltpu.VMEM((_FEAT_DIM, _SB), jnp.float32),
            pltpu.VMEM((_FEAT_DIM, _SB), jnp.float32),
            pltpu.VMEM((_BATCH,), jnp.int32),
            pltpu.VMEM((_STAGE + 1, _FEAT_DIM), jnp.float32),
            pltpu.SMEM((_SB_PER_W + 2,), jnp.int32),
            pltpu.SMEM((_SB_PER_W + 2,), jnp.int32),
            pltpu.SMEM((_SB_PER_W + 2,), jnp.int32),
            pltpu.SMEM((8,), jnp.int32),
            pltpu.SemaphoreType.DMA,
            pltpu.SemaphoreType.DMA,
            pltpu.SemaphoreType.DMA,
        ],
    )(labels, centersT)


_BLK = 1024
_GRID = _BATCH // _BLK


def _k2_body(featT_ref, e_ref, lab_ref, tail_ref, out_ref):
    i = pl.program_id(0)

    @pl.when(i == 0)
    def _():
        out_ref[0, 0] = 0.0

    ft = featT_ref[...]
    e = e_ref[...]
    lab = lab_ref[...]
    tail = tail_ref[...]

    is_tail = lab >= _TAIL
    rowids = lax.broadcasted_iota(jnp.int32, (1, _FEAT_DIM), 1) + _TAIL
    oh = jnp.where(lab == rowids, 1.0, 0.0)
    texp = jax.lax.dot(oh, tail, precision=jax.lax.Precision.HIGHEST)
    e_sel = jnp.where(is_tail, texp, e)

    m = jax.lax.dot(ft, e_sel, precision=jax.lax.Precision.HIGHEST)
    eye = jnp.where(
        lax.broadcasted_iota(jnp.int32, (_FEAT_DIM, _FEAT_DIM), 0)
        == lax.broadcasted_iota(jnp.int32, (_FEAT_DIM, _FEAT_DIM), 1),
        1.0, 0.0)
    cross = jnp.sum(m * eye)
    total = jnp.sum(ft * ft) + jnp.sum(e_sel * e_sel) - 2.0 * cross
    out_ref[0, 0] += total * (_LAMBDA_C / float(_BATCH * _FEAT_DIM))


def _k2(featT, e, labels2d, tail):
    return pl.pallas_call(
        _k2_body,
        grid=(_GRID,),
        in_specs=[
            pl.BlockSpec((_FEAT_DIM, _BLK), lambda i: (0, i)),
            pl.BlockSpec((_BLK, _FEAT_DIM), lambda i: (i, 0)),
            pl.BlockSpec((_BLK, 1), lambda i: (i, 0)),
            pl.BlockSpec((_FEAT_DIM, _FEAT_DIM), lambda i: (0, 0)),
        ],
        out_specs=pl.BlockSpec(memory_space=pltpu.SMEM),
        out_shape=jax.ShapeDtypeStruct((1, 1), jnp.float32),
    )(featT, e, labels2d, tail)


@jax.jit
def kernel(features, labels, centers):
    lab = labels.astype(jnp.int32)
    centersT = centers.T
    e = _k1(lab, centersT)
    tail = lax.slice(centersT, (0, _TAIL), (_FEAT_DIM, _NUM_CLASSES))
    tail = jnp.transpose(tail)
    loss = _k2(features.T, e, lab.reshape(_BATCH, 1), tail)
    return loss[0, 0]


# speedup vs baseline: 1.1249x; 1.1249x over previous
import jax
import jax.numpy as jnp
from jax import lax
from jax.experimental import pallas as pl
from jax.experimental.pallas import tpu as pltpu
from jax.experimental.pallas import tpu_sc as plsc

_NUM_CLASSES = 1000000
_FEAT_DIM = 64
_BATCH = 16384
_LAMBDA_C = 1.0

_NC = 2
_NS = 16
_NW = _NC * _NS
_ROWS_W = _BATCH // _NW
_CHUNK = 128
_NCHUNK = _ROWS_W // _CHUNK
_LANES = 16
_PACK = 2 * _FEAT_DIM


def _cl_body(feat_hbm, lab_hbm, cent2_hbm, out_hbm,
             lab_v, idx_v, feat_v, gath_a, gath_b, acc_v,
             semf, sem0, sem1):
    wid = lax.axis_index("s") * _NC + lax.axis_index("c")
    base = wid * _ROWS_W
    gbufs = [gath_a, gath_b]
    sems = [sem0, sem1]
    iota = lax.iota(jnp.int32, _LANES)

    pltpu.sync_copy(lab_hbm.at[pl.ds(base, _ROWS_W)], lab_v)
    fcp = pltpu.async_copy(feat_hbm.at[pl.ds(base, _ROWS_W), :], feat_v,
                           semf)

    def mk_idx(g, carry):
        vec = lab_v[pl.ds(g * _LANES, _LANES)]
        idx_v[g // 8, pl.ds((g % 8) * _LANES, _LANES)] = vec >> 1
        return carry
    lax.fori_loop(0, _ROWS_W // _LANES, mk_idx, 0)

    def fire(c):
        return pltpu.async_copy(cent2_hbm.at[idx_v.at[c]], gbufs[c % 2],
                                sems[c % 2])

    fire(0)
    fire(1)
    fcp.wait()

    acc = jnp.zeros((_LANES,), jnp.float32)
    for c in range(_NCHUNK):
        pltpu.make_async_copy(cent2_hbm.at[pl.ds(0, _CHUNK)], gbufs[c % 2],
                              sems[c % 2]).wait()
        gbuf = gbufs[c % 2]

        def grp_step(g, acc, c=c, gbuf=gbuf):
            vec = lab_v[pl.ds(c * _CHUNK + g * _LANES, _LANES)]
            a = acc
            for l in range(_LANES):
                row = g * _LANES + l
                odd = jnp.broadcast_to(
                    (vec[l] & 1).astype(jnp.float32), (_LANES,))
                for j in range(_FEAT_DIM // _LANES):
                    f = feat_v[c * _CHUNK + row, pl.ds(j * _LANES, _LANES)]
                    ce0 = gbuf[row, pl.ds(j * _LANES, _LANES)]
                    ce1 = gbuf[row, pl.ds(_FEAT_DIM + j * _LANES, _LANES)]
                    ce = ce0 + (ce1 - ce0) * odd
                    dd = f - ce
                    a = a + dd * dd
            return a

        acc = lax.fori_loop(0, _CHUNK // _LANES, grp_step, acc)
        if c + 2 < _NCHUNK:
            fire(c + 2)

    acc_v[...] = acc * (_LAMBDA_C / float(_BATCH * _FEAT_DIM))
    pltpu.sync_copy(acc_v, out_hbm.at[wid])


@jax.jit
def kernel(features, labels, centers):
    mesh = plsc.VectorSubcoreMesh(core_axis_name="c", subcore_axis_name="s")
    cent2 = centers.reshape(_NUM_CLASSES // 2, _PACK)
    partials = pl.kernel(
        _cl_body,
        mesh=mesh,
        out_type=jax.ShapeDtypeStruct((_NW, _LANES), jnp.float32),
        scratch_types=[
            pltpu.VMEM((_ROWS_W,), jnp.int32),
            pltpu.VMEM((_NCHUNK, _CHUNK), jnp.int32),
            pltpu.VMEM((_ROWS_W, _FEAT_DIM), jnp.float32),
            pltpu.VMEM((_CHUNK, _PACK), jnp.float32),
            pltpu.VMEM((_CHUNK, _PACK), jnp.float32),
            pltpu.VMEM((_LANES,), jnp.float32),
            pltpu.SemaphoreType.DMA,
            pltpu.SemaphoreType.DMA,
            pltpu.SemaphoreType.DMA,
        ],
    )(features, labels.astype(jnp.int32), cent2)
    return jnp.sum(partials)


# speedup vs baseline: 1.9055x; 1.6940x over previous
import jax
import jax.numpy as jnp
from jax import lax
from jax.experimental import pallas as pl
from jax.experimental.pallas import tpu as pltpu
from jax.experimental.pallas import tpu_sc as plsc

_NUM_CLASSES = 1000000
_FEAT_DIM = 64
_BATCH = 16384
_LAMBDA_C = 1.0

_NC = 2
_NS = 16
_NW = _NC * _NS
_ROWS_W = _BATCH // _NW
_CHUNK = 128
_NCHUNK = _ROWS_W // _CHUNK
_LANES = 16
_GROUPS = _FEAT_DIM // _LANES


def _cl_body(feat_hbm, lab_hbm, cent_hbm, out_hbm,
             lab_v, feat_a, feat_b, rows_v, acc_v,
             semf, sem0, sem1, sem2, sem3):
    wid = lax.axis_index("s") * _NC + lax.axis_index("c")
    base = wid * _ROWS_W
    row_sems = [sem0, sem1, sem2, sem3]
    fbufs = [feat_a, feat_b]

    pltpu.sync_copy(lab_hbm.at[pl.ds(base, _ROWS_W)], lab_v)

    fcps = [pltpu.async_copy(feat_hbm.at[pl.ds(base, _CHUNK), :],
                             feat_a, semf)]

    for c in range(_NCHUNK):
        def issue(g, carry, c=c):
            vec = lab_v[pl.ds(c * _CHUNK + g * _LANES, _LANES)]
            for l in range(_LANES):
                r = vec[l]
                pltpu.async_copy(
                    cent_hbm.at[pl.ds(r, 1), :],
                    rows_v.at[pl.ds(c * _CHUNK + g * _LANES + l, 1), :],
                    row_sems[c])
            return carry
        lax.fori_loop(0, _CHUNK // _LANES, issue, 0)

    acc = jnp.zeros((_LANES,), jnp.float32)
    for c in range(_NCHUNK):
        if c + 1 < _NCHUNK:
            fcps.append(
                pltpu.async_copy(
                    feat_hbm.at[pl.ds(base + (c + 1) * _CHUNK, _CHUNK), :],
                    fbufs[(c + 1) % 2], semf))
        fcps[c].wait()
        pltpu.make_async_copy(cent_hbm.at[pl.ds(0, _CHUNK), :],
                              rows_v.at[pl.ds(c * _CHUNK, _CHUNK), :],
                              row_sems[c]).wait()

        fbuf = fbufs[c % 2]

        def row_step(i, acc, c=c, fbuf=fbuf):
            for j in range(_GROUPS):
                f = fbuf[i, pl.ds(j * _LANES, _LANES)]
                ce = rows_v[c * _CHUNK + i, pl.ds(j * _LANES, _LANES)]
                d = f - ce
                acc = acc + d * d
            return acc

        acc = lax.fori_loop(0, _CHUNK, row_step, acc)

    acc_v[...] = acc * (_LAMBDA_C / float(_BATCH * _FEAT_DIM))
    pltpu.sync_copy(acc_v, out_hbm.at[wid])


@jax.jit
def kernel(features, labels, centers):
    mesh = plsc.VectorSubcoreMesh(core_axis_name="c", subcore_axis_name="s")
    partials = pl.kernel(
        _cl_body,
        mesh=mesh,
        out_type=jax.ShapeDtypeStruct((_NW, _LANES), jnp.float32),
        scratch_types=[
            pltpu.VMEM((_ROWS_W,), jnp.int32),
            pltpu.VMEM((_CHUNK, _FEAT_DIM), jnp.float32),
            pltpu.VMEM((_CHUNK, _FEAT_DIM), jnp.float32),
            pltpu.VMEM((_ROWS_W, _FEAT_DIM), jnp.float32),
            pltpu.VMEM((_LANES,), jnp.float32),
            pltpu.SemaphoreType.DMA,
            pltpu.SemaphoreType.DMA,
            pltpu.SemaphoreType.DMA,
            pltpu.SemaphoreType.DMA,
            pltpu.SemaphoreType.DMA,
        ],
    )(features, labels.astype(jnp.int32), centers)
    return jnp.sum(partials)
